# Initial kernel scaffold; baseline (speedup 1.0000x reference)
import jax
import jax.numpy as jnp
from jax.experimental import pallas as pl


def kernel(emb_s, emb_t, device):
    raise NotImplementedError("write your pallas kernel here")


# speedup vs baseline: 45.8597x; 45.8597x over previous
import jax
import jax.numpy as jnp
from jax.experimental import pallas as pl
from jax.experimental.pallas import tpu as pltpu

_G = 4
_N = 4096
_K = 32
_R = 256


def _mask_kernel(s_ref, t_ref, o_ref, candp_ref, candn_ref):
    nb = pl.program_id(1)

    @pl.when(nb == 0)
    def _compute_candidates():
        iota = jax.lax.broadcasted_iota(jnp.int32, (1, _N), 1)
        tv = t_ref[0, :, :]
        for sgn, cand_ref in ((1.0, candp_ref), (-1.0, candn_ref)):
            v = tv * sgn
            for k in range(_K + 1):
                m = jnp.max(v)
                idx = jnp.min(jnp.where(v == m, iota, _N))
                cand_ref[k] = idx
                v = jnp.where(iota == idx, -jnp.inf, v)

    ci = jax.lax.broadcasted_iota(jnp.int32, (1, _N), 1)
    ri = nb * _R + jax.lax.broadcasted_iota(jnp.int32, (_R, 1), 0)

    basep_j = jnp.zeros((1, _N), jnp.float32)
    basen_j = jnp.zeros((1, _N), jnp.float32)
    inp_i = jnp.zeros((_R, 1), jnp.float32)
    inn_i = jnp.zeros((_R, 1), jnp.float32)
    for k in range(_K):
        cp = candp_ref[k]
        cn = candn_ref[k]
        basep_j += (ci == cp).astype(jnp.float32)
        basen_j += (ci == cn).astype(jnp.float32)
        inp_i += (ri == cp).astype(jnp.float32)
        inn_i += (ri == cn).astype(jnp.float32)
    extrap_j = (ci == candp_ref[_K]).astype(jnp.float32)
    extran_j = (ci == candn_ref[_K]).astype(jnp.float32)

    neq = (ci != ri).astype(jnp.float32)
    pos = basep_j * neq + extrap_j * inp_i
    neg = basen_j * neq + extran_j * inn_i
    s_blk = s_ref[0, :, :]
    o_ref[0, :, :] = jnp.where(s_blk > 0.0, pos, neg)


def kernel(emb_s, emb_t, device):
    del device
    nb = _N // _R
    return pl.pallas_call(
        _mask_kernel,
        grid=(_G, nb),
        in_specs=[
            pl.BlockSpec((1, _R, 1), lambda g, b: (g, b, 0)),
            pl.BlockSpec((1, 1, _N), lambda g, b: (g, 0, 0)),
        ],
        out_specs=pl.BlockSpec((1, _R, _N), lambda g, b: (g, b, 0)),
        out_shape=jax.ShapeDtypeStruct((_G, _N, _N), jnp.float32),
        scratch_shapes=[
            pltpu.SMEM((_K + 1,), jnp.int32),
            pltpu.SMEM((_K + 1,), jnp.int32),
        ],
    )(emb_s, emb_t)
